# baseline (speedup 1.0000x reference)
import jax
import jax.numpy as jnp
from jax.experimental import pallas as pl
from jax.experimental.pallas import tpu as pltpu

_SEQ_BLK = 2048
_BH_BLK = 8


def _copy_body(pos_ref, cache_ref, update_ref, out_ref):
    j = pl.program_id(1)
    pos = pos_ref[0]
    row0 = j * _SEQ_BLK
    upd_len = update_ref.shape[1]
    d = cache_ref.shape[2]

    out_ref[...] = cache_ref[...]

    overlaps = (pos < row0 + _SEQ_BLK) & (row0 < pos + upd_len)

    @pl.when(overlaps)
    def _patch():
        win = jnp.clip(pos - row0, 0, _SEQ_BLK - upd_len)
        rel = (
            jax.lax.broadcasted_iota(jnp.int32, (upd_len, d), 0)
            + row0
            + win
            - pos
        )
        for b in range(_BH_BLK):
            val = out_ref[b, pl.ds(win, upd_len), :]
            for k in range(upd_len):
                val = jnp.where(rel == k, update_ref[b, k, :][None, :], val)
            out_ref[b, pl.ds(win, upd_len), :] = val


def kernel(cache, update, pos):
    b, h, s, d = cache.shape
    upd_len = update.shape[-2]
    cache3 = cache.reshape(b * h, s, d)
    update3 = update.reshape(b * h, upd_len, d)
    pos_arr = jnp.asarray(pos, jnp.int32).reshape((1,))

    grid = (b * h // _BH_BLK, s // _SEQ_BLK)
    out3 = pl.pallas_call(
        _copy_body,
        grid_spec=pltpu.PrefetchScalarGridSpec(
            num_scalar_prefetch=1,
            grid=grid,
            in_specs=[
                pl.BlockSpec(
                    (_BH_BLK, _SEQ_BLK, d), lambda i, j, pos_ref: (i, j, 0),
                ),
                pl.BlockSpec(
                    (_BH_BLK, upd_len, d), lambda i, j, pos_ref: (i, 0, 0)
                ),
            ],
            out_specs=pl.BlockSpec(
                (_BH_BLK, _SEQ_BLK, d), lambda i, j, pos_ref: (i, j, 0),
            ),
        ),
        out_shape=jax.ShapeDtypeStruct((b * h, s, d), cache.dtype),
    )(pos_arr, cache3, update3)
    return out3.reshape(b, h, s, d)
